# Initial kernel scaffold; baseline (speedup 1.0000x reference)
import jax
import jax.numpy as jnp
from jax.experimental import pallas as pl


def kernel(h_paper, h_author, h_subject, Wsrc_a, Wdst_a, al_a, ar_a, b_a, Wsrc_s, Wdst_s, al_s, ar_s, b_s, W_fc, b_fc, att, nbr_author, nbr_subject):
    raise NotImplementedError("write your pallas kernel here")


# speedup vs baseline: 4.5809x; 4.5809x over previous
import functools

import jax
import jax.numpy as jnp
from jax import lax
from jax.experimental import pallas as pl
from jax.experimental.pallas import tpu as pltpu
from jax.experimental.pallas import tpu_sc as plsc

_N = 50000
_D = 128
_S_A = 8
_S_S = 4
_N_PAD = 50176
_NC = 2
_NS = 16
_NW = _NC * _NS


@functools.lru_cache(maxsize=None)
def _make_sc_gather(S, K):
    B = _N_PAD * S
    bpw = B // _NW
    nch = bpw // 128
    nbody = nch // K
    assert nch % K == 0

    mesh = plsc.VectorSubcoreMesh(
        core_axis_name="c", subcore_axis_name="s",
        num_cores=_NC, num_subcores=_NS)

    @functools.partial(
        pl.kernel,
        mesh=mesh,
        out_type=jax.ShapeDtypeStruct((B, _D), jnp.float32),
        scratch_types=[
            pltpu.VMEM((nch, 128), jnp.int32),
            pltpu.VMEM((K * 128, _D), jnp.float32),
            pltpu.SemaphoreType.DMA,
        ],
    )
    def gather(table, idx, out, idx_v, stage, sem):
        wid = lax.axis_index("s") * _NC + lax.axis_index("c")
        pltpu.sync_copy(idx.at[wid], idx_v)

        def body(i, carry):
            cps = [
                pltpu.async_copy(
                    table.at[idx_v.at[i * K + b]],
                    stage.at[pl.ds(b * 128, 128)],
                    sem,
                )
                for b in range(K)
            ]
            for c in cps:
                c.wait()
            pltpu.sync_copy(
                stage, out.at[pl.ds(wid * bpw + i * (K * 128), K * 128)]
            )
            return carry

        lax.fori_loop(0, nbody, body, 0)

    return gather


def _proj_kernel(ha_ref, hs_ref, wa_ref, ws_ref, fa_ref, fs_ref):
    dn = (((1,), (1,)), ((), ()))
    fa_ref[...] = lax.dot_general(
        ha_ref[...], wa_ref[...], dn, preferred_element_type=jnp.float32)
    fs_ref[...] = lax.dot_general(
        hs_ref[...], ws_ref[...], dn, preferred_element_type=jnp.float32)


def _gat_tail_kernel(fsg_ref, hp_ref, wdst_ref, al_ref, ar_ref, b_ref, out_ref):
    fsg = fsg_ref[...]
    el_g = jnp.sum(fsg * al_ref[...], axis=-1)
    wv = lax.dot_general(
        ar_ref[...], wdst_ref[...], (((1,), (0,)), ((), ())),
        preferred_element_type=jnp.float32)
    er = jnp.sum(hp_ref[...] * wv, axis=-1, keepdims=True)
    e = el_g + er
    e = jnp.where(e >= 0.0, e, 0.2 * e)
    m = jnp.max(e, axis=-1, keepdims=True)
    p = jnp.exp(e - m)
    alpha = p / jnp.sum(p, axis=-1, keepdims=True)
    out_ref[...] = jnp.sum(alpha[:, :, None] * fsg, axis=1) + b_ref[...]


def _beta_kernel(e0_ref, e1_ref, wfc_ref, bfc_ref, att_ref, l0_ref, l1_ref):
    @pl.when(pl.program_id(0) == 0)
    def _():
        l0_ref[0, 0] = 0.0
        l1_ref[0, 0] = 0.0

    dn = (((1,), (1,)), ((), ()))

    def part(e):
        t = jnp.tanh(
            lax.dot_general(e, wfc_ref[...], dn,
                            preferred_element_type=jnp.float32)
            + bfc_ref[...])
        return jnp.sum(t * att_ref[...])

    l0_ref[0, 0] += part(e0_ref[...])
    l1_ref[0, 0] += part(e1_ref[...])


def _combine_kernel(l0_ref, l1_ref, e0_ref, e1_ref, z_ref):
    d = (l1_ref[0, 0] - l0_ref[0, 0]) * (1.0 / _N)
    e0 = e0_ref[...]
    beta0 = 1.0 / (1.0 + jnp.exp(jnp.full(e0.shape, d, jnp.float32)))
    z_ref[...] = beta0 * e0 + (1.0 - beta0) * e1_ref[...]


def _flat_idx(nbr, S):
    nbr = nbr.astype(jnp.int32)
    pad = jnp.zeros((_N_PAD - _N, S), jnp.int32)
    return jnp.concatenate([nbr, pad], axis=0).reshape(_NW, -1, 128)


def kernel(h_paper, h_author, h_subject,
           Wsrc_a, Wdst_a, al_a, ar_a, b_a,
           Wsrc_s, Wdst_s, al_s, ar_s, b_s,
           W_fc, b_fc, att,
           nbr_author, nbr_subject):
    BN1 = 1000
    fs_a, fs_s = pl.pallas_call(
        _proj_kernel,
        grid=(_N // BN1,),
        in_specs=[
            pl.BlockSpec((BN1, _D), lambda i: (i, 0)),
            pl.BlockSpec((BN1, _D), lambda i: (i, 0)),
            pl.BlockSpec((_D, _D), lambda i: (0, 0)),
            pl.BlockSpec((_D, _D), lambda i: (0, 0)),
        ],
        out_specs=[pl.BlockSpec((BN1, _D), lambda i: (i, 0))] * 2,
        out_shape=[jax.ShapeDtypeStruct((_N, _D), jnp.float32)] * 2,
    )(h_author, h_subject, Wsrc_a, Wsrc_s)

    fsg_a = _make_sc_gather(_S_A, 7)(fs_a, _flat_idx(nbr_author, _S_A))
    fsg_s = _make_sc_gather(_S_S, 7)(fs_s, _flat_idx(nbr_subject, _S_S))

    BN = 400
    grid = (_N // BN,)

    def gat_tail(fsg, S, Wdst, al, ar, b):
        return pl.pallas_call(
            _gat_tail_kernel,
            grid=grid,
            in_specs=[
                pl.BlockSpec((BN, S, _D), lambda i: (i, 0, 0)),
                pl.BlockSpec((BN, _D), lambda i: (i, 0)),
                pl.BlockSpec((_D, _D), lambda i: (0, 0)),
                pl.BlockSpec((1, 1, _D), lambda i: (0, 0, 0)),
                pl.BlockSpec((1, _D), lambda i: (0, 0)),
                pl.BlockSpec((1, _D), lambda i: (0, 0)),
            ],
            out_specs=pl.BlockSpec((BN, _D), lambda i: (i, 0)),
            out_shape=jax.ShapeDtypeStruct((_N, _D), jnp.float32),
        )(fsg.reshape(_N_PAD, S, _D), h_paper, Wdst,
          al.reshape(1, 1, _D), ar.reshape(1, _D), b.reshape(1, _D))

    e0 = gat_tail(fsg_a, _S_A, Wdst_a, al_a, ar_a, b_a)
    e1 = gat_tail(fsg_s, _S_S, Wdst_s, al_s, ar_s, b_s)

    l0, l1 = pl.pallas_call(
        _beta_kernel,
        grid=grid,
        in_specs=[
            pl.BlockSpec((BN, _D), lambda i: (i, 0)),
            pl.BlockSpec((BN, _D), lambda i: (i, 0)),
            pl.BlockSpec((_D, _D), lambda i: (0, 0)),
            pl.BlockSpec((1, _D), lambda i: (0, 0)),
            pl.BlockSpec((1, _D), lambda i: (0, 0)),
        ],
        out_specs=[pl.BlockSpec(memory_space=pltpu.SMEM)] * 2,
        out_shape=[jax.ShapeDtypeStruct((1, 1), jnp.float32)] * 2,
    )(e0, e1, W_fc, b_fc.reshape(1, _D), att.reshape(1, _D))

    z = pl.pallas_call(
        _combine_kernel,
        grid=grid,
        in_specs=[
            pl.BlockSpec(memory_space=pltpu.SMEM),
            pl.BlockSpec(memory_space=pltpu.SMEM),
            pl.BlockSpec((BN, _D), lambda i: (i, 0)),
            pl.BlockSpec((BN, _D), lambda i: (i, 0)),
        ],
        out_specs=pl.BlockSpec((BN, _D), lambda i: (i, 0)),
        out_shape=jax.ShapeDtypeStruct((_N, _D), jnp.float32),
    )(l0, l1, e0, e1)

    return z
